# baseline (speedup 1.0000x reference)
import functools

import jax
import jax.numpy as jnp
from jax import lax
from jax.experimental import pallas as pl
from jax.experimental.pallas import tpu as pltpu
from jax.experimental.pallas import tpu_sc as plsc

F32 = jnp.float32


def _dg(a, b):
    return lax.dot_general(a, b, (((1,), (1,)), ((), ())),
                           preferred_element_type=F32)


def _dgn(a, b):
    return lax.dot_general(a, b, (((1,), (0,)), ((), ())),
                           preferred_element_type=F32)


def _silu(x):
    return x * jax.nn.sigmoid(x)


def _emb_body(nf, cp, eW, eb, W1a, W1b, h_o, Ga_o, Gb_o):
    h = _dg(nf[...], eW[...]) + eb[...]
    h_o[...] = h
    cpv = cp[...]
    z = jnp.zeros((cpv.shape[0], 112), F32)
    Ga_o[...] = jnp.concatenate([_dg(h, W1a[...]), cpv, z], axis=1)
    Gb_o[...] = jnp.concatenate([_dg(h, W1b[...]), cpv, z], axis=1)


def _emb_call(nf, cp, eW, eb, W1a, W1b, N, BN):
    grid = (N // BN,)
    full = lambda shp: pl.BlockSpec(shp, lambda i: (0, 0))
    blk = pl.BlockSpec((BN, 128), lambda i: (i, 0))
    blk16 = pl.BlockSpec((BN, 16), lambda i: (i, 0))
    blk256 = pl.BlockSpec((BN, 256), lambda i: (i, 0))
    return pl.pallas_call(
        _emb_body,
        grid=grid,
        in_specs=[blk, blk16, full((128, 128)), full((1, 128)),
                  full((128, 128)), full((128, 128))],
        out_specs=[blk, blk256, blk256],
        out_shape=[jax.ShapeDtypeStruct((N, 128), F32),
                   jax.ShapeDtypeStruct((N, 256), F32),
                   jax.ShapeDtypeStruct((N, 256), F32)],
    )(nf, cp, eW, eb, W1a, W1b)


def _edge_body(mpre, cd, W2, b2, w1c, b1, cW1, cb1, cW2, cb2, msg_o, cupd_o):
    cdv = cd[...]
    r2 = jnp.sum(cdv * cdv, axis=1, keepdims=True)
    radial = jnp.sqrt(r2)
    m = _silu(mpre[...] + radial * w1c[...] + b1[...])
    msgs = _dg(m, W2[...]) + b2[...]
    msg_o[...] = msgs
    cw = _silu(_dg(msgs, cW1[...]) + cb1[...])
    w = jnp.sum(cw * cW2[...], axis=1, keepdims=True) + cb2[...]
    cupd_o[...] = w * cdv / (radial + 1e-8)


def _edge_call(mpre, cdiff, p, E, BE):
    grid = (E // BE,)
    full = lambda shp: pl.BlockSpec(shp, lambda i: tuple(0 for _ in shp))
    blk128 = pl.BlockSpec((BE, 128), lambda i: (i, 0))
    return pl.pallas_call(
        _edge_body,
        grid=grid,
        in_specs=[blk128, blk128,
                  full((128, 128)), full((1, 128)), full((1, 128)),
                  full((1, 128)), full((64, 128)), full((1, 64)),
                  full((1, 64)), full((1, 1))],
        out_specs=[blk128, blk128],
        out_shape=[jax.ShapeDtypeStruct((E, 128), F32),
                   jax.ShapeDtypeStruct((E, 128), F32)],
    )(mpre, cdiff,
      p['msg_W2'], p['msg_b2'].reshape(1, 128),
      p['msg_W1'][:, 256].reshape(1, 128), p['msg_b1'].reshape(1, 128),
      p['coord_W1'], p['coord_b1'].reshape(1, 64),
      p['coord_W2'], p['coord_b2'].reshape(1, 1))


def _node_body(h, aggA, aggB, cp, cgA, cgB, W1h, W1a, b1, W2, b2,
               nW1a, nW1b, hout, cout, Ga_o, Gb_o):
    hv = h[...]
    agg = aggA[0] + aggB[0]
    n = _silu(_dg(hv, W1h[...]) + _dg(agg, W1a[...]) + b1[...])
    ho = hv + _dg(n, W2[...]) + b2[...]
    hout[...] = ho
    cnew = cp[...] + cgA[0][:, :16] + cgB[0][:, :16]
    cout[...] = cnew
    z = jnp.zeros((cnew.shape[0], 112), F32)
    Ga_o[...] = jnp.concatenate([_dg(ho, nW1a[...]), cnew, z], axis=1)
    Gb_o[...] = jnp.concatenate([_dg(ho, nW1b[...]), cnew, z], axis=1)


def _node_call(h, aggp, cp, caggp, p, nextp, N, BN):
    grid = (N // BN,)
    full = lambda shp: pl.BlockSpec(shp, lambda i: tuple(0 for _ in shp))
    blk128 = pl.BlockSpec((BN, 128), lambda i: (i, 0))
    prt0 = pl.BlockSpec((1, BN, 128), lambda i: (0, i, 0))
    prt1 = pl.BlockSpec((1, BN, 128), lambda i: (1, i, 0))
    blk16 = pl.BlockSpec((BN, 16), lambda i: (i, 0))
    blk256 = pl.BlockSpec((BN, 256), lambda i: (i, 0))
    return pl.pallas_call(
        _node_body,
        grid=grid,
        in_specs=[blk128, prt0, prt1, blk16, prt0, prt1,
                  full((128, 128)), full((128, 128)), full((1, 128)),
                  full((128, 128)), full((1, 128)),
                  full((128, 128)), full((128, 128))],
        out_specs=[blk128, blk16, blk256, blk256],
        out_shape=[jax.ShapeDtypeStruct((N, 128), F32),
                   jax.ShapeDtypeStruct((N, 16), F32),
                   jax.ShapeDtypeStruct((N, 256), F32),
                   jax.ShapeDtypeStruct((N, 256), F32)],
    )(h, aggp, aggp, cp, caggp, caggp,
      p['node_W1'][:, :128], p['node_W1'][:, 128:],
      p['node_b1'].reshape(1, 128), p['node_W2'],
      p['node_b2'].reshape(1, 128),
      nextp['msg_W1'][:, :128], nextp['msg_W1'][:, 128:256])


def _node3_body(h, aggA, aggB, W1h, W1a, b1, W2, b2,
                Wq, bq, Wk, bk, Wv, bv, hout, qo, ko, vo):
    hv = h[...]
    agg = aggA[0] + aggB[0]
    n = _silu(_dg(hv, W1h[...]) + _dg(agg, W1a[...]) + b1[...])
    ho = hv + _dg(n, W2[...]) + b2[...]
    hout[...] = ho
    qo[...] = _dg(ho, Wq[...]) + bq[...]
    ko[...] = _dg(ho, Wk[...]) + bk[...]
    vo[...] = _dg(ho, Wv[...]) + bv[...]


def _node3_call(h, aggp, p, params, N, BN):
    grid = (N // BN,)
    full = lambda shp: pl.BlockSpec(shp, lambda i: tuple(0 for _ in shp))
    blk128 = pl.BlockSpec((BN, 128), lambda i: (i, 0))
    prt0 = pl.BlockSpec((1, BN, 128), lambda i: (0, i, 0))
    prt1 = pl.BlockSpec((1, BN, 128), lambda i: (1, i, 0))
    return pl.pallas_call(
        _node3_body,
        grid=grid,
        in_specs=[blk128, prt0, prt1,
                  full((128, 128)), full((128, 128)), full((1, 128)),
                  full((128, 128)), full((1, 128)),
                  full((128, 128)), full((1, 128)),
                  full((128, 128)), full((1, 128)),
                  full((128, 128)), full((1, 128))],
        out_specs=[blk128, blk128, blk128, blk128],
        out_shape=[jax.ShapeDtypeStruct((N, 128), F32)] * 4,
    )(h, aggp, aggp,
      p['node_W1'][:, :128], p['node_W1'][:, 128:],
      p['node_b1'].reshape(1, 128), p['node_W2'],
      p['node_b2'].reshape(1, 128),
      params['Wq'], params['bq'].reshape(1, 128),
      params['Wk'], params['bk'].reshape(1, 128),
      params['Wv'], params['bv'].reshape(1, 128))


def _attn_body(q, k, v, o, *, BQ, N, KC, DH):
    scale = 1.0 / (DH ** 0.5)
    nchunk = N // KC
    qh = q[0] * scale
    m0 = jnp.full((BQ, 1), -1e30, F32)
    l0 = jnp.zeros((BQ, 1), F32)
    a0 = jnp.zeros((BQ, DH), F32)

    def step(c, carry):
        m, l, acc = carry
        kc = k[0, pl.ds(c * KC, KC), :]
        vc = v[0, pl.ds(c * KC, KC), :]
        s = _dg(qh, kc)
        mnew = jnp.maximum(m, jnp.max(s, axis=1, keepdims=True))
        pexp = jnp.exp(s - mnew)
        corr = jnp.exp(m - mnew)
        l = l * corr + jnp.sum(pexp, axis=1, keepdims=True)
        acc = acc * corr + _dgn(pexp, vc)
        return mnew, l, acc

    m, l, acc = lax.fori_loop(0, nchunk, step, (m0, l0, a0))
    o[0] = acc / l


def _attn_call(q, k, v, N, BQ):
    qh = jnp.transpose(q.reshape(N, 8, 16), (1, 0, 2))
    kh = jnp.transpose(k.reshape(N, 8, 16), (1, 0, 2))
    vh = jnp.transpose(v.reshape(N, 8, 16), (1, 0, 2))
    grid = (8, N // BQ)
    full = pl.BlockSpec((1, N, 16), lambda h, i: (h, 0, 0))
    blk = pl.BlockSpec((1, BQ, 16), lambda h, i: (h, i, 0))
    body = functools.partial(_attn_body, BQ=BQ, N=N, KC=2000, DH=16)
    out = pl.pallas_call(
        body,
        grid=grid,
        in_specs=[blk, full, full],
        out_specs=blk,
        out_shape=jax.ShapeDtypeStruct((8, N, 16), F32),
    )(qh, kh, vh)
    return jnp.transpose(out, (1, 0, 2)).reshape(N, 128)


def _final_body(h, at, Wo, bo, g, b, sW1, sb1, sW2, sb2, asg,
                hf, site, sA, sB, cA, cB):
    i = pl.program_id(0)
    x = h[...] + _dg(at[...], Wo[...]) + bo[...]
    mu = jnp.mean(x, axis=1, keepdims=True)
    xc = x - mu
    var = jnp.mean(xc * xc, axis=1, keepdims=True)
    hn = xc / jnp.sqrt(var + 1e-5) * g[...] + b[...]
    hf[...] = hn
    s = jax.nn.relu(_dg(hn, sW1[...]) + sb1[...])
    site[...] = jax.nn.sigmoid(
        jnp.sum(s * sW2[...], axis=1, keepdims=True) + sb2[...])
    av = asg[...]
    ma = (av == 0).astype(F32)
    mb = (av == 1).astype(F32)

    @pl.when(i == 0)
    def _():
        sA[...] = jnp.zeros_like(sA)
        sB[...] = jnp.zeros_like(sB)
        cA[...] = jnp.zeros_like(cA)
        cB[...] = jnp.zeros_like(cB)

    sA[...] += jnp.sum(hn * ma, axis=0, keepdims=True)
    sB[...] += jnp.sum(hn * mb, axis=0, keepdims=True)
    cA[...] += jnp.sum(ma).reshape(1, 1)
    cB[...] += jnp.sum(mb).reshape(1, 1)


def _final_call(h, at, params, asg2d, N, BN):
    grid = (N // BN,)
    full = lambda shp: pl.BlockSpec(shp, lambda i: tuple(0 for _ in shp))
    blk128 = pl.BlockSpec((BN, 128), lambda i: (i, 0))
    blk1 = pl.BlockSpec((BN, 1), lambda i: (i, 0))
    acc128 = pl.BlockSpec((1, 128), lambda i: (0, 0))
    acc1 = pl.BlockSpec((1, 1), lambda i: (0, 0))
    return pl.pallas_call(
        _final_body,
        grid=grid,
        in_specs=[blk128, blk128,
                  full((128, 128)), full((1, 128)), full((1, 128)),
                  full((1, 128)), full((64, 128)), full((1, 64)),
                  full((1, 64)), full((1, 1)), blk1],
        out_specs=[blk128, blk1, acc128, acc128, acc1, acc1],
        out_shape=[jax.ShapeDtypeStruct((N, 128), F32),
                   jax.ShapeDtypeStruct((N, 1), F32),
                   jax.ShapeDtypeStruct((1, 128), F32),
                   jax.ShapeDtypeStruct((1, 128), F32),
                   jax.ShapeDtypeStruct((1, 1), F32),
                   jax.ShapeDtypeStruct((1, 1), F32)],
    )(h, at, params['Wo'], params['bo'].reshape(1, 128),
      params['ln_g'].reshape(1, 128), params['ln_b'].reshape(1, 128),
      params['site_W1'], params['site_b1'].reshape(1, 64),
      params['site_W2'], params['site_b2'].reshape(1, 1), asg2d)


def _ppi_body(sA, sB, cA, cB, W1a, W1b, b1, W2, b2, o):
    ha = sA[...] / jnp.maximum(cA[...], 1.0)
    hb = sB[...] / jnp.maximum(cB[...], 1.0)
    z = jax.nn.relu(_dg(ha, W1a[...]) + _dg(hb, W1b[...]) + b1[...])
    o[...] = jax.nn.sigmoid(
        jnp.sum(z * W2[...], axis=1, keepdims=True) + b2[...])


def _ppi_call(sA, sB, cA, cB, params):
    full = lambda shp: pl.BlockSpec(shp, lambda: tuple(0 for _ in shp))
    return pl.pallas_call(
        _ppi_body,
        in_specs=[full((1, 128)), full((1, 128)), full((1, 1)), full((1, 1)),
                  full((128, 128)), full((128, 128)), full((1, 128)),
                  full((1, 128)), full((1, 1))],
        out_specs=full((1, 1)),
        out_shape=jax.ShapeDtypeStruct((1, 1), F32),
    )(sA, sB, cA, cB, params['ppi_W1'][:, :128], params['ppi_W1'][:, 128:],
      params['ppi_b1'].reshape(1, 128), params['ppi_W2'],
      params['ppi_b2'].reshape(1, 1))


_NW = 32
_CHUNK = 128


def _sc_mesh():
    return plsc.VectorSubcoreMesh(core_axis_name="c", subcore_axis_name="s")


def _sc_gather(Ga, Gb, row1d, col1d, E):
    nchunk = E // _CHUNK
    niter = (nchunk + _NW - 1) // _NW

    @functools.partial(
        pl.kernel, mesh=_sc_mesh(),
        out_type=[jax.ShapeDtypeStruct((E, 128), F32),
                  jax.ShapeDtypeStruct((E, 128), F32)],
        scratch_types=[pltpu.VMEM((_CHUNK,), jnp.int32),
                       pltpu.VMEM((_CHUNK,), jnp.int32),
                       pltpu.VMEM((_CHUNK, 256), F32),
                       pltpu.VMEM((_CHUNK, 256), F32),
                       pltpu.VMEM((_CHUNK, 128), F32),
                       pltpu.SemaphoreType.DMA,
                       pltpu.SemaphoreType.DMA],
    )
    def k(Ga_h, Gb_h, row_h, col_h, mpre_h, cdiff_h,
          ir, ic, bA, bB, bC, s1, s2):
        wid = lax.axis_index("s") * 2 + lax.axis_index("c")
        zv = jnp.zeros((16,), F32)

        @pl.loop(0, _CHUNK)
        def _(i):
            for c8 in range(1, 8):
                bC[i, pl.ds(c8 * 16, 16)] = zv

        @pl.loop(0, niter)
        def _(jj):
            r = wid + _NW * jj

            @pl.when(r < nchunk)
            def _():
                pltpu.sync_copy(row_h.at[pl.ds(r * _CHUNK, _CHUNK)], ir)
                pltpu.sync_copy(col_h.at[pl.ds(r * _CHUNK, _CHUNK)], ic)
                c1 = pltpu.async_copy(Ga_h.at[ir], bA, s1)
                c2 = pltpu.async_copy(Gb_h.at[ic], bB, s2)
                c1.wait()
                c2.wait()

                @pl.loop(0, _CHUNK)
                def _(i):
                    s0 = pl.ds(128, 16)
                    bC[i, pl.ds(0, 16)] = bA[i, s0] - bB[i, s0]
                    for c8 in range(8):
                        sl = pl.ds(c8 * 16, 16)
                        bA[i, sl] = bA[i, sl] + bB[i, sl]

                pltpu.sync_copy(bA.at[:, pl.ds(0, 128)],
                                mpre_h.at[pl.ds(r * _CHUNK, _CHUNK)])
                pltpu.sync_copy(bC, cdiff_h.at[pl.ds(r * _CHUNK, _CHUNK)])

    return k(Ga, Gb, row1d, col1d)


def _sc_scatter(msgs, cupd, row1d, z128, NP):
    E = msgs.shape[0]
    nchunk = E // _CHUNK
    niter = (nchunk + _NW - 1) // _NW
    rows_per_sub = NP // 16

    @functools.partial(
        pl.kernel, mesh=_sc_mesh(),
        out_type=[jax.ShapeDtypeStruct((2, NP, 128), F32),
                  jax.ShapeDtypeStruct((2, NP, 128), F32)],
        scratch_types=[pltpu.VMEM_SHARED((NP, 128), F32),
                       pltpu.VMEM((_CHUNK,), jnp.int32),
                       pltpu.VMEM((_CHUNK, 128), F32)],
    )
    def k(msgs_h, cupd_h, row_h, z128_h, agg_h, cagg_h, sh_acc, ir, mb):
        c = lax.axis_index("c")
        s = lax.axis_index("s")
        wid = s * 2 + c
        base = s * rows_per_sub
        for src_h, dst_h in ((msgs_h, agg_h), (cupd_h, cagg_h)):
            pltpu.sync_copy(z128_h.at[pl.ds(base, rows_per_sub)],
                            sh_acc.at[pl.ds(base, rows_per_sub)])
            plsc.subcore_barrier()

            @pl.loop(0, niter)
            def _(jj):
                r = wid + _NW * jj

                @pl.when(r < nchunk)
                def _():
                    pltpu.sync_copy(row_h.at[pl.ds(r * _CHUNK, _CHUNK)], ir)
                    pltpu.sync_copy(src_h.at[pl.ds(r * _CHUNK, _CHUNK)], mb)
                    pltpu.sync_copy(mb, sh_acc.at[ir], add=True)

            plsc.subcore_barrier()
            pltpu.sync_copy(sh_acc.at[pl.ds(base, rows_per_sub)],
                            dst_h.at[c, pl.ds(base, rows_per_sub)])
            plsc.subcore_barrier()

    return k(msgs, cupd, row1d, z128)


def kernel(node_features, coords, edges, protein_assignment, params):
    N, D = node_features.shape
    E = edges.shape[1]
    NP = ((N + 127) // 128) * 128
    row = edges[0].astype(jnp.int32)
    col = edges[1].astype(jnp.int32)
    cpad = jnp.zeros((N, 16), F32).at[:, :3].set(coords.astype(F32))
    z128 = jnp.zeros((NP, 128), F32)

    layers = params['layers']
    h, Ga, Gb = _emb_call(node_features, cpad, params['emb_W'],
                          params['emb_b'].reshape(1, 128),
                          layers[0]['msg_W1'][:, :128],
                          layers[0]['msg_W1'][:, 128:256], N, 1000)

    q = k_ = v = None
    for li in range(len(layers)):
        p = layers[li]
        mpre, cdiff = _sc_gather(Ga, Gb, row, col, E)
        msgs, cupd = _edge_call(mpre, cdiff, p, E, 1000)
        aggp, caggp = _sc_scatter(msgs, cupd, row, z128, NP)
        if li + 1 < len(layers):
            h, cpad, Ga, Gb = _node_call(h, aggp, cpad, caggp, p,
                                         layers[li + 1], N, 1000)
        else:
            h, q, k_, v = _node3_call(h, aggp, p, params, N, 1000)

    attn = _attn_call(q, k_, v, N, 400)
    asg2d = protein_assignment.astype(jnp.int32).reshape(N, 1)
    hf, site, sA, sB, cA, cB = _final_call(h, attn, params, asg2d, N, 400)
    ppi = _ppi_call(sA, sB, cA, cB, params)
    return ppi.reshape(()), site.reshape(N), hf


# speedup vs baseline: 1.1956x; 1.1956x over previous
import functools

import jax
import jax.numpy as jnp
from jax import lax
from jax.experimental import pallas as pl
from jax.experimental.pallas import tpu as pltpu
from jax.experimental.pallas import tpu_sc as plsc

F32 = jnp.float32


def _dg(a, b):
    return lax.dot_general(a, b, (((1,), (1,)), ((), ())),
                           preferred_element_type=F32)


def _dgn(a, b):
    return lax.dot_general(a, b, (((1,), (0,)), ((), ())),
                           preferred_element_type=F32)


def _silu(x):
    return x * jax.nn.sigmoid(x)


def _emb_body(nf, cp, eW, eb, W1a, W1b, h_o, Ga_o, Gb_o):
    h = _dg(nf[...], eW[...]) + eb[...]
    h_o[...] = h
    cpv = cp[...]
    z = jnp.zeros((cpv.shape[0], 112), F32)
    Ga_o[...] = jnp.concatenate([_dg(h, W1a[...]), cpv, z], axis=1)
    Gb_o[...] = jnp.concatenate([_dg(h, W1b[...]), cpv, z], axis=1)


def _emb_call(nf, cp, eW, eb, W1a, W1b, N, BN):
    grid = (N // BN,)
    full = lambda shp: pl.BlockSpec(shp, lambda i: (0, 0))
    blk = pl.BlockSpec((BN, 128), lambda i: (i, 0))
    blk16 = pl.BlockSpec((BN, 16), lambda i: (i, 0))
    blk256 = pl.BlockSpec((BN, 256), lambda i: (i, 0))
    return pl.pallas_call(
        _emb_body,
        grid=grid,
        in_specs=[blk, blk16, full((128, 128)), full((1, 128)),
                  full((128, 128)), full((128, 128))],
        out_specs=[blk, blk256, blk256],
        out_shape=[jax.ShapeDtypeStruct((N, 128), F32),
                   jax.ShapeDtypeStruct((N, 256), F32),
                   jax.ShapeDtypeStruct((N, 256), F32)],
    )(nf, cp, eW, eb, W1a, W1b)


def _edge_body(mpre, cd, W2, b2, w1c, b1, cW1, cb1, cW2, cb2, msg_o, cupd_o):
    cdv = cd[...]
    r2 = jnp.sum(cdv * cdv, axis=1, keepdims=True)
    radial = jnp.sqrt(r2)
    m = _silu(mpre[...] + radial * w1c[...] + b1[...])
    msgs = _dg(m, W2[...]) + b2[...]
    msg_o[...] = msgs
    cw = _silu(_dg(msgs, cW1[...]) + cb1[...])
    w = jnp.sum(cw * cW2[...], axis=1, keepdims=True) + cb2[...]
    cupd_o[...] = w * cdv / (radial + 1e-8)


def _edge_call(mpre, cdiff, p, E, BE):
    grid = (E // BE,)
    full = lambda shp: pl.BlockSpec(shp, lambda i: tuple(0 for _ in shp))
    blk128 = pl.BlockSpec((BE, 128), lambda i: (i, 0))
    return pl.pallas_call(
        _edge_body,
        grid=grid,
        in_specs=[blk128, blk128,
                  full((128, 128)), full((1, 128)), full((1, 128)),
                  full((1, 128)), full((64, 128)), full((1, 64)),
                  full((1, 64)), full((1, 1))],
        out_specs=[blk128, blk128],
        out_shape=[jax.ShapeDtypeStruct((E, 128), F32),
                   jax.ShapeDtypeStruct((E, 128), F32)],
    )(mpre, cdiff,
      p['msg_W2'], p['msg_b2'].reshape(1, 128),
      p['msg_W1'][:, 256].reshape(1, 128), p['msg_b1'].reshape(1, 128),
      p['coord_W1'], p['coord_b1'].reshape(1, 64),
      p['coord_W2'], p['coord_b2'].reshape(1, 1))


def _node_body(h, aggA, aggB, cp, cgA, cgB, W1h, W1a, b1, W2, b2,
               nW1a, nW1b, hout, cout, Ga_o, Gb_o):
    hv = h[...]
    agg = aggA[0] + aggB[0]
    n = _silu(_dg(hv, W1h[...]) + _dg(agg, W1a[...]) + b1[...])
    ho = hv + _dg(n, W2[...]) + b2[...]
    hout[...] = ho
    cnew = cp[...] + cgA[0][:, :16] + cgB[0][:, :16]
    cout[...] = cnew
    z = jnp.zeros((cnew.shape[0], 112), F32)
    Ga_o[...] = jnp.concatenate([_dg(ho, nW1a[...]), cnew, z], axis=1)
    Gb_o[...] = jnp.concatenate([_dg(ho, nW1b[...]), cnew, z], axis=1)


def _node_call(h, aggp, cp, caggp, p, nextp, N, BN):
    grid = (N // BN,)
    full = lambda shp: pl.BlockSpec(shp, lambda i: tuple(0 for _ in shp))
    blk128 = pl.BlockSpec((BN, 128), lambda i: (i, 0))
    prt0 = pl.BlockSpec((1, BN, 128), lambda i: (0, i, 0))
    prt1 = pl.BlockSpec((1, BN, 128), lambda i: (1, i, 0))
    blk16 = pl.BlockSpec((BN, 16), lambda i: (i, 0))
    blk256 = pl.BlockSpec((BN, 256), lambda i: (i, 0))
    return pl.pallas_call(
        _node_body,
        grid=grid,
        in_specs=[blk128, prt0, prt1, blk16, prt0, prt1,
                  full((128, 128)), full((128, 128)), full((1, 128)),
                  full((128, 128)), full((1, 128)),
                  full((128, 128)), full((128, 128))],
        out_specs=[blk128, blk16, blk256, blk256],
        out_shape=[jax.ShapeDtypeStruct((N, 128), F32),
                   jax.ShapeDtypeStruct((N, 16), F32),
                   jax.ShapeDtypeStruct((N, 256), F32),
                   jax.ShapeDtypeStruct((N, 256), F32)],
    )(h, aggp, aggp, cp, caggp, caggp,
      p['node_W1'][:, :128], p['node_W1'][:, 128:],
      p['node_b1'].reshape(1, 128), p['node_W2'],
      p['node_b2'].reshape(1, 128),
      nextp['msg_W1'][:, :128], nextp['msg_W1'][:, 128:256])


def _node3_body(h, aggA, aggB, W1h, W1a, b1, W2, b2,
                Wq, bq, Wk, bk, Wv, bv, hout, qo, ko, vo):
    hv = h[...]
    agg = aggA[0] + aggB[0]
    n = _silu(_dg(hv, W1h[...]) + _dg(agg, W1a[...]) + b1[...])
    ho = hv + _dg(n, W2[...]) + b2[...]
    hout[...] = ho
    qo[...] = _dg(ho, Wq[...]) + bq[...]
    ko[...] = _dg(ho, Wk[...]) + bk[...]
    vo[...] = _dg(ho, Wv[...]) + bv[...]


def _node3_call(h, aggp, p, params, N, BN):
    grid = (N // BN,)
    full = lambda shp: pl.BlockSpec(shp, lambda i: tuple(0 for _ in shp))
    blk128 = pl.BlockSpec((BN, 128), lambda i: (i, 0))
    prt0 = pl.BlockSpec((1, BN, 128), lambda i: (0, i, 0))
    prt1 = pl.BlockSpec((1, BN, 128), lambda i: (1, i, 0))
    return pl.pallas_call(
        _node3_body,
        grid=grid,
        in_specs=[blk128, prt0, prt1,
                  full((128, 128)), full((128, 128)), full((1, 128)),
                  full((128, 128)), full((1, 128)),
                  full((128, 128)), full((1, 128)),
                  full((128, 128)), full((1, 128)),
                  full((128, 128)), full((1, 128))],
        out_specs=[blk128, blk128, blk128, blk128],
        out_shape=[jax.ShapeDtypeStruct((N, 128), F32)] * 4,
    )(h, aggp, aggp,
      p['node_W1'][:, :128], p['node_W1'][:, 128:],
      p['node_b1'].reshape(1, 128), p['node_W2'],
      p['node_b2'].reshape(1, 128),
      params['Wq'], params['bq'].reshape(1, 128),
      params['Wk'], params['bk'].reshape(1, 128),
      params['Wv'], params['bv'].reshape(1, 128))


def _attn_body(q, k, v, o, *, BQ, N, KC, DH):
    scale = 1.0 / (DH ** 0.5)
    nchunk = N // KC
    qh = q[0] * scale
    m0 = jnp.full((BQ, 1), -1e30, F32)
    l0 = jnp.zeros((BQ, 1), F32)
    a0 = jnp.zeros((BQ, DH), F32)

    def step(c, carry):
        m, l, acc = carry
        kc = k[0, pl.ds(c * KC, KC), :]
        vc = v[0, pl.ds(c * KC, KC), :]
        s = _dg(qh, kc)
        mnew = jnp.maximum(m, jnp.max(s, axis=1, keepdims=True))
        pexp = jnp.exp(s - mnew)
        corr = jnp.exp(m - mnew)
        l = l * corr + jnp.sum(pexp, axis=1, keepdims=True)
        acc = acc * corr + _dgn(pexp, vc)
        return mnew, l, acc

    m, l, acc = lax.fori_loop(0, nchunk, step, (m0, l0, a0))
    o[0] = acc / l


def _attn_call(q, k, v, N, BQ):
    qh = jnp.transpose(q.reshape(N, 8, 16), (1, 0, 2))
    kh = jnp.transpose(k.reshape(N, 8, 16), (1, 0, 2))
    vh = jnp.transpose(v.reshape(N, 8, 16), (1, 0, 2))
    grid = (8, N // BQ)
    full = pl.BlockSpec((1, N, 16), lambda h, i: (h, 0, 0))
    blk = pl.BlockSpec((1, BQ, 16), lambda h, i: (h, i, 0))
    body = functools.partial(_attn_body, BQ=BQ, N=N, KC=2000, DH=16)
    out = pl.pallas_call(
        body,
        grid=grid,
        in_specs=[blk, full, full],
        out_specs=blk,
        out_shape=jax.ShapeDtypeStruct((8, N, 16), F32),
    )(qh, kh, vh)
    return jnp.transpose(out, (1, 0, 2)).reshape(N, 128)


def _final_body(h, at, Wo, bo, g, b, sW1, sb1, sW2, sb2, asg,
                hf, site, sA, sB, cA, cB):
    i = pl.program_id(0)
    x = h[...] + _dg(at[...], Wo[...]) + bo[...]
    mu = jnp.mean(x, axis=1, keepdims=True)
    xc = x - mu
    var = jnp.mean(xc * xc, axis=1, keepdims=True)
    hn = xc / jnp.sqrt(var + 1e-5) * g[...] + b[...]
    hf[...] = hn
    s = jax.nn.relu(_dg(hn, sW1[...]) + sb1[...])
    site[...] = jax.nn.sigmoid(
        jnp.sum(s * sW2[...], axis=1, keepdims=True) + sb2[...])
    av = asg[...]
    ma = (av == 0).astype(F32)
    mb = (av == 1).astype(F32)

    @pl.when(i == 0)
    def _():
        sA[...] = jnp.zeros_like(sA)
        sB[...] = jnp.zeros_like(sB)
        cA[...] = jnp.zeros_like(cA)
        cB[...] = jnp.zeros_like(cB)

    sA[...] += jnp.sum(hn * ma, axis=0, keepdims=True)
    sB[...] += jnp.sum(hn * mb, axis=0, keepdims=True)
    cA[...] += jnp.sum(ma).reshape(1, 1)
    cB[...] += jnp.sum(mb).reshape(1, 1)


def _final_call(h, at, params, asg2d, N, BN):
    grid = (N // BN,)
    full = lambda shp: pl.BlockSpec(shp, lambda i: tuple(0 for _ in shp))
    blk128 = pl.BlockSpec((BN, 128), lambda i: (i, 0))
    blk1 = pl.BlockSpec((BN, 1), lambda i: (i, 0))
    acc128 = pl.BlockSpec((1, 128), lambda i: (0, 0))
    acc1 = pl.BlockSpec((1, 1), lambda i: (0, 0))
    return pl.pallas_call(
        _final_body,
        grid=grid,
        in_specs=[blk128, blk128,
                  full((128, 128)), full((1, 128)), full((1, 128)),
                  full((1, 128)), full((64, 128)), full((1, 64)),
                  full((1, 64)), full((1, 1)), blk1],
        out_specs=[blk128, blk1, acc128, acc128, acc1, acc1],
        out_shape=[jax.ShapeDtypeStruct((N, 128), F32),
                   jax.ShapeDtypeStruct((N, 1), F32),
                   jax.ShapeDtypeStruct((1, 128), F32),
                   jax.ShapeDtypeStruct((1, 128), F32),
                   jax.ShapeDtypeStruct((1, 1), F32),
                   jax.ShapeDtypeStruct((1, 1), F32)],
    )(h, at, params['Wo'], params['bo'].reshape(1, 128),
      params['ln_g'].reshape(1, 128), params['ln_b'].reshape(1, 128),
      params['site_W1'], params['site_b1'].reshape(1, 64),
      params['site_W2'], params['site_b2'].reshape(1, 1), asg2d)


def _ppi_body(sA, sB, cA, cB, W1a, W1b, b1, W2, b2, o):
    ha = sA[...] / jnp.maximum(cA[...], 1.0)
    hb = sB[...] / jnp.maximum(cB[...], 1.0)
    z = jax.nn.relu(_dg(ha, W1a[...]) + _dg(hb, W1b[...]) + b1[...])
    o[...] = jax.nn.sigmoid(
        jnp.sum(z * W2[...], axis=1, keepdims=True) + b2[...])


def _ppi_call(sA, sB, cA, cB, params):
    full = lambda shp: pl.BlockSpec(shp, lambda: tuple(0 for _ in shp))
    return pl.pallas_call(
        _ppi_body,
        in_specs=[full((1, 128)), full((1, 128)), full((1, 1)), full((1, 1)),
                  full((128, 128)), full((128, 128)), full((1, 128)),
                  full((1, 128)), full((1, 1))],
        out_specs=full((1, 1)),
        out_shape=jax.ShapeDtypeStruct((1, 1), F32),
    )(sA, sB, cA, cB, params['ppi_W1'][:, :128], params['ppi_W1'][:, 128:],
      params['ppi_b1'].reshape(1, 128), params['ppi_W2'],
      params['ppi_b2'].reshape(1, 1))


_NW = 32
_CHUNK = 128


def _sc_mesh():
    return plsc.VectorSubcoreMesh(core_axis_name="c", subcore_axis_name="s")


def _sc_gather(Ga, Gb, row1d, col1d, E):
    nchunk = E // _CHUNK
    niter = (nchunk + _NW - 1) // _NW

    @functools.partial(
        pl.kernel, mesh=_sc_mesh(),
        out_type=[jax.ShapeDtypeStruct((E, 128), F32),
                  jax.ShapeDtypeStruct((E, 128), F32)],
        scratch_types=[pltpu.VMEM((_CHUNK,), jnp.int32),
                       pltpu.VMEM((_CHUNK,), jnp.int32),
                       pltpu.VMEM((_CHUNK, 256), F32),
                       pltpu.VMEM((_CHUNK, 256), F32),
                       pltpu.VMEM((_CHUNK, 128), F32),
                       pltpu.SemaphoreType.DMA,
                       pltpu.SemaphoreType.DMA],
    )
    def k(Ga_h, Gb_h, row_h, col_h, mpre_h, cdiff_h,
          ir, ic, bA, bB, bC, s1, s2):
        wid = lax.axis_index("s") * 2 + lax.axis_index("c")
        zv = jnp.zeros((16,), F32)

        @pl.loop(0, _CHUNK)
        def _(i):
            for c8 in range(1, 8):
                bC[i, pl.ds(c8 * 16, 16)] = zv

        @pl.loop(0, niter)
        def _(jj):
            r = wid + _NW * jj

            @pl.when(r < nchunk)
            def _():
                pltpu.sync_copy(row_h.at[pl.ds(r * _CHUNK, _CHUNK)], ir)
                pltpu.sync_copy(col_h.at[pl.ds(r * _CHUNK, _CHUNK)], ic)
                c1 = pltpu.async_copy(Ga_h.at[ir], bA, s1)
                c2 = pltpu.async_copy(Gb_h.at[ic], bB, s2)
                c1.wait()
                c2.wait()

                @pl.loop(0, _CHUNK)
                def _(i):
                    s0 = pl.ds(128, 16)
                    bC[i, pl.ds(0, 16)] = bA[i, s0] - bB[i, s0]
                    for c8 in range(8):
                        sl = pl.ds(c8 * 16, 16)
                        bA[i, sl] = bA[i, sl] + bB[i, sl]

                pltpu.sync_copy(bA.at[:, pl.ds(0, 128)],
                                mpre_h.at[pl.ds(r * _CHUNK, _CHUNK)])
                pltpu.sync_copy(bC, cdiff_h.at[pl.ds(r * _CHUNK, _CHUNK)])

    return k(Ga, Gb, row1d, col1d)


def _sc_scatter(msgs, cupd, row1d, z128, NP):
    E = msgs.shape[0]
    nchunk = E // _CHUNK
    niter = (nchunk + _NW - 1) // _NW
    rows_per_sub = NP // 16

    @functools.partial(
        pl.kernel, mesh=_sc_mesh(),
        out_type=[jax.ShapeDtypeStruct((2, NP, 128), F32),
                  jax.ShapeDtypeStruct((2, NP, 128), F32)],
        scratch_types=[pltpu.VMEM_SHARED((NP, 128), F32),
                       pltpu.VMEM((_CHUNK,), jnp.int32),
                       pltpu.VMEM((_CHUNK, 128), F32)],
    )
    def k(msgs_h, cupd_h, row_h, z128_h, agg_h, cagg_h, sh_acc, ir, mb):
        c = lax.axis_index("c")
        s = lax.axis_index("s")
        wid = s * 2 + c
        base = s * rows_per_sub
        for src_h, dst_h in ((msgs_h, agg_h), (cupd_h, cagg_h)):
            pltpu.sync_copy(z128_h.at[pl.ds(base, rows_per_sub)],
                            sh_acc.at[pl.ds(base, rows_per_sub)])
            plsc.subcore_barrier()

            @pl.loop(0, niter)
            def _(jj):
                r = wid + _NW * jj

                @pl.when(r < nchunk)
                def _():
                    pltpu.sync_copy(row_h.at[pl.ds(r * _CHUNK, _CHUNK)], ir)
                    pltpu.sync_copy(src_h.at[pl.ds(r * _CHUNK, _CHUNK)], mb)
                    pltpu.sync_copy(mb, sh_acc.at[ir], add=True)

            plsc.subcore_barrier()
            pltpu.sync_copy(sh_acc.at[pl.ds(base, rows_per_sub)],
                            dst_h.at[c, pl.ds(base, rows_per_sub)])
            plsc.subcore_barrier()

    return k(msgs, cupd, row1d, z128)


def kernel(node_features, coords, edges, protein_assignment, params):
    N, D = node_features.shape
    E = edges.shape[1]
    NP = ((N + 127) // 128) * 128
    row = edges[0].astype(jnp.int32)
    col = edges[1].astype(jnp.int32)
    cpad = jnp.zeros((N, 16), F32).at[:, :3].set(coords.astype(F32))
    z128 = jnp.zeros((NP, 128), F32)

    layers = params['layers']
    h, Ga, Gb = _emb_call(node_features, cpad, params['emb_W'],
                          params['emb_b'].reshape(1, 128),
                          layers[0]['msg_W1'][:, :128],
                          layers[0]['msg_W1'][:, 128:256], N, 1000)

    q = k_ = v = None
    for li in range(len(layers)):
        p = layers[li]
        mpre, cdiff = _sc_gather(Ga, Gb, row, col, E)
        msgs, cupd = _edge_call(mpre, cdiff, p, E, 4000)
        aggp, caggp = _sc_scatter(msgs, cupd, row, z128, NP)
        if li + 1 < len(layers):
            h, cpad, Ga, Gb = _node_call(h, aggp, cpad, caggp, p,
                                         layers[li + 1], N, 1000)
        else:
            h, q, k_, v = _node3_call(h, aggp, p, params, N, 1000)

    attn = _attn_call(q, k_, v, N, 1000)
    asg2d = protein_assignment.astype(jnp.int32).reshape(N, 1)
    hf, site, sA, sB, cA, cB = _final_call(h, attn, params, asg2d, N, 400)
    ppi = _ppi_call(sA, sB, cA, cB, params)
    return ppi.reshape(()), site.reshape(N), hf


# speedup vs baseline: 1.2061x; 1.0088x over previous
import functools

import jax
import jax.numpy as jnp
from jax import lax
from jax.experimental import pallas as pl
from jax.experimental.pallas import tpu as pltpu
from jax.experimental.pallas import tpu_sc as plsc

F32 = jnp.float32


def _dg(a, b):
    return lax.dot_general(a, b, (((1,), (1,)), ((), ())),
                           preferred_element_type=F32)


def _dgn(a, b):
    return lax.dot_general(a, b, (((1,), (0,)), ((), ())),
                           preferred_element_type=F32)


def _silu(x):
    return x * jax.nn.sigmoid(x)


def _emb_body(nf, cp, eW, eb, W1a, W1b, h_o, Ga_o, Gb_o):
    h = _dg(nf[...], eW[...]) + eb[...]
    h_o[...] = h
    cpv = cp[...]
    z = jnp.zeros((cpv.shape[0], 112), F32)
    Ga_o[...] = jnp.concatenate([_dg(h, W1a[...]), cpv, z], axis=1)
    Gb_o[...] = jnp.concatenate([_dg(h, W1b[...]), cpv, z], axis=1)


def _emb_call(nf, cp, eW, eb, W1a, W1b, N, BN):
    grid = (N // BN,)
    full = lambda shp: pl.BlockSpec(shp, lambda i: (0, 0))
    blk = pl.BlockSpec((BN, 128), lambda i: (i, 0))
    blk16 = pl.BlockSpec((BN, 16), lambda i: (i, 0))
    blk256 = pl.BlockSpec((BN, 256), lambda i: (i, 0))
    return pl.pallas_call(
        _emb_body,
        grid=grid,
        in_specs=[blk, blk16, full((128, 128)), full((1, 128)),
                  full((128, 128)), full((128, 128))],
        out_specs=[blk, blk256, blk256],
        out_shape=[jax.ShapeDtypeStruct((N, 128), F32),
                   jax.ShapeDtypeStruct((N, 256), F32),
                   jax.ShapeDtypeStruct((N, 256), F32)],
    )(nf, cp, eW, eb, W1a, W1b)


def _edge_body(mpre, cd, W2, b2, w1c, b1, cW1, cb1, cW2, cb2, msg_o, cupd_o):
    cdv = cd[...]
    r2 = jnp.sum(cdv * cdv, axis=1, keepdims=True)
    radial = jnp.sqrt(r2)
    m = _silu(mpre[...] + radial * w1c[...] + b1[...])
    msgs = _dg(m, W2[...]) + b2[...]
    msg_o[...] = msgs
    cw = _silu(_dg(msgs, cW1[...]) + cb1[...])
    w = jnp.sum(cw * cW2[...], axis=1, keepdims=True) + cb2[...]
    cupd_o[...] = w * cdv / (radial + 1e-8)


def _edge_call(mpre, cdiff, p, E, BE):
    grid = (E // BE,)
    full = lambda shp: pl.BlockSpec(shp, lambda i: tuple(0 for _ in shp))
    blk128 = pl.BlockSpec((BE, 128), lambda i: (i, 0))
    return pl.pallas_call(
        _edge_body,
        grid=grid,
        in_specs=[blk128, blk128,
                  full((128, 128)), full((1, 128)), full((1, 128)),
                  full((1, 128)), full((64, 128)), full((1, 64)),
                  full((1, 64)), full((1, 1))],
        out_specs=[blk128, blk128],
        out_shape=[jax.ShapeDtypeStruct((E, 128), F32),
                   jax.ShapeDtypeStruct((E, 128), F32)],
    )(mpre, cdiff,
      p['msg_W2'], p['msg_b2'].reshape(1, 128),
      p['msg_W1'][:, 256].reshape(1, 128), p['msg_b1'].reshape(1, 128),
      p['coord_W1'], p['coord_b1'].reshape(1, 64),
      p['coord_W2'], p['coord_b2'].reshape(1, 1))


def _node_body(h, aggA, aggB, cp, cgA, cgB, W1h, W1a, b1, W2, b2,
               nW1a, nW1b, hout, cout, Ga_o, Gb_o):
    hv = h[...]
    agg = aggA[0] + aggB[0]
    n = _silu(_dg(hv, W1h[...]) + _dg(agg, W1a[...]) + b1[...])
    ho = hv + _dg(n, W2[...]) + b2[...]
    hout[...] = ho
    cnew = cp[...] + cgA[0][:, :16] + cgB[0][:, :16]
    cout[...] = cnew
    z = jnp.zeros((cnew.shape[0], 112), F32)
    Ga_o[...] = jnp.concatenate([_dg(ho, nW1a[...]), cnew, z], axis=1)
    Gb_o[...] = jnp.concatenate([_dg(ho, nW1b[...]), cnew, z], axis=1)


def _node_call(h, aggp, cp, caggp, p, nextp, N, BN):
    grid = (N // BN,)
    full = lambda shp: pl.BlockSpec(shp, lambda i: tuple(0 for _ in shp))
    blk128 = pl.BlockSpec((BN, 128), lambda i: (i, 0))
    prt0 = pl.BlockSpec((1, BN, 128), lambda i: (0, i, 0))
    prt1 = pl.BlockSpec((1, BN, 128), lambda i: (1, i, 0))
    blk16 = pl.BlockSpec((BN, 16), lambda i: (i, 0))
    blk256 = pl.BlockSpec((BN, 256), lambda i: (i, 0))
    return pl.pallas_call(
        _node_body,
        grid=grid,
        in_specs=[blk128, prt0, prt1, blk16, prt0, prt1,
                  full((128, 128)), full((128, 128)), full((1, 128)),
                  full((128, 128)), full((1, 128)),
                  full((128, 128)), full((128, 128))],
        out_specs=[blk128, blk16, blk256, blk256],
        out_shape=[jax.ShapeDtypeStruct((N, 128), F32),
                   jax.ShapeDtypeStruct((N, 16), F32),
                   jax.ShapeDtypeStruct((N, 256), F32),
                   jax.ShapeDtypeStruct((N, 256), F32)],
    )(h, aggp, aggp, cp, caggp, caggp,
      p['node_W1'][:, :128], p['node_W1'][:, 128:],
      p['node_b1'].reshape(1, 128), p['node_W2'],
      p['node_b2'].reshape(1, 128),
      nextp['msg_W1'][:, :128], nextp['msg_W1'][:, 128:256])


def _node3_body(h, aggA, aggB, W1h, W1a, b1, W2, b2,
                Wq, bq, Wk, bk, Wv, bv, hout, qo, ko, vo):
    hv = h[...]
    agg = aggA[0] + aggB[0]
    n = _silu(_dg(hv, W1h[...]) + _dg(agg, W1a[...]) + b1[...])
    ho = hv + _dg(n, W2[...]) + b2[...]
    hout[...] = ho
    qo[...] = _dg(ho, Wq[...]) + bq[...]
    ko[...] = _dg(ho, Wk[...]) + bk[...]
    vo[...] = _dg(ho, Wv[...]) + bv[...]


def _node3_call(h, aggp, p, params, N, BN):
    grid = (N // BN,)
    full = lambda shp: pl.BlockSpec(shp, lambda i: tuple(0 for _ in shp))
    blk128 = pl.BlockSpec((BN, 128), lambda i: (i, 0))
    prt0 = pl.BlockSpec((1, BN, 128), lambda i: (0, i, 0))
    prt1 = pl.BlockSpec((1, BN, 128), lambda i: (1, i, 0))
    return pl.pallas_call(
        _node3_body,
        grid=grid,
        in_specs=[blk128, prt0, prt1,
                  full((128, 128)), full((128, 128)), full((1, 128)),
                  full((128, 128)), full((1, 128)),
                  full((128, 128)), full((1, 128)),
                  full((128, 128)), full((1, 128)),
                  full((128, 128)), full((1, 128))],
        out_specs=[blk128, blk128, blk128, blk128],
        out_shape=[jax.ShapeDtypeStruct((N, 128), F32)] * 4,
    )(h, aggp, aggp,
      p['node_W1'][:, :128], p['node_W1'][:, 128:],
      p['node_b1'].reshape(1, 128), p['node_W2'],
      p['node_b2'].reshape(1, 128),
      params['Wq'], params['bq'].reshape(1, 128),
      params['Wk'], params['bk'].reshape(1, 128),
      params['Wv'], params['bv'].reshape(1, 128))


def _attn_body(q, k, v, o, *, BQ, N, KC, DH):
    scale = 1.0 / (DH ** 0.5)
    nchunk = N // KC
    qh = q[0] * scale
    m0 = jnp.full((BQ, 1), -1e30, F32)
    l0 = jnp.zeros((BQ, 1), F32)
    a0 = jnp.zeros((BQ, DH), F32)

    def step(c, carry):
        m, l, acc = carry
        kc = k[0, pl.ds(c * KC, KC), :]
        vc = v[0, pl.ds(c * KC, KC), :]
        s = _dg(qh, kc)
        mnew = jnp.maximum(m, jnp.max(s, axis=1, keepdims=True))
        pexp = jnp.exp(s - mnew)
        corr = jnp.exp(m - mnew)
        l = l * corr + jnp.sum(pexp, axis=1, keepdims=True)
        acc = acc * corr + _dgn(pexp, vc)
        return mnew, l, acc

    m, l, acc = lax.fori_loop(0, nchunk, step, (m0, l0, a0))
    o[0] = acc / l


def _attn_call(q, k, v, N, BQ):
    qh = jnp.transpose(q.reshape(N, 8, 16), (1, 0, 2))
    kh = jnp.transpose(k.reshape(N, 8, 16), (1, 0, 2))
    vh = jnp.transpose(v.reshape(N, 8, 16), (1, 0, 2))
    grid = (8, N // BQ)
    full = pl.BlockSpec((1, N, 16), lambda h, i: (h, 0, 0))
    blk = pl.BlockSpec((1, BQ, 16), lambda h, i: (h, i, 0))
    body = functools.partial(_attn_body, BQ=BQ, N=N, KC=2000, DH=16)
    out = pl.pallas_call(
        body,
        grid=grid,
        in_specs=[blk, full, full],
        out_specs=blk,
        out_shape=jax.ShapeDtypeStruct((8, N, 16), F32),
    )(qh, kh, vh)
    return jnp.transpose(out, (1, 0, 2)).reshape(N, 128)


def _final_body(h, at, Wo, bo, g, b, sW1, sb1, sW2, sb2, asg,
                hf, site, sA, sB, cA, cB):
    i = pl.program_id(0)
    x = h[...] + _dg(at[...], Wo[...]) + bo[...]
    mu = jnp.mean(x, axis=1, keepdims=True)
    xc = x - mu
    var = jnp.mean(xc * xc, axis=1, keepdims=True)
    hn = xc / jnp.sqrt(var + 1e-5) * g[...] + b[...]
    hf[...] = hn
    s = jax.nn.relu(_dg(hn, sW1[...]) + sb1[...])
    site[...] = jax.nn.sigmoid(
        jnp.sum(s * sW2[...], axis=1, keepdims=True) + sb2[...])
    av = asg[...]
    ma = (av == 0).astype(F32)
    mb = (av == 1).astype(F32)

    @pl.when(i == 0)
    def _():
        sA[...] = jnp.zeros_like(sA)
        sB[...] = jnp.zeros_like(sB)
        cA[...] = jnp.zeros_like(cA)
        cB[...] = jnp.zeros_like(cB)

    sA[...] += jnp.sum(hn * ma, axis=0, keepdims=True)
    sB[...] += jnp.sum(hn * mb, axis=0, keepdims=True)
    cA[...] += jnp.sum(ma).reshape(1, 1)
    cB[...] += jnp.sum(mb).reshape(1, 1)


def _final_call(h, at, params, asg2d, N, BN):
    grid = (N // BN,)
    full = lambda shp: pl.BlockSpec(shp, lambda i: tuple(0 for _ in shp))
    blk128 = pl.BlockSpec((BN, 128), lambda i: (i, 0))
    blk1 = pl.BlockSpec((BN, 1), lambda i: (i, 0))
    acc128 = pl.BlockSpec((1, 128), lambda i: (0, 0))
    acc1 = pl.BlockSpec((1, 1), lambda i: (0, 0))
    return pl.pallas_call(
        _final_body,
        grid=grid,
        in_specs=[blk128, blk128,
                  full((128, 128)), full((1, 128)), full((1, 128)),
                  full((1, 128)), full((64, 128)), full((1, 64)),
                  full((1, 64)), full((1, 1)), blk1],
        out_specs=[blk128, blk1, acc128, acc128, acc1, acc1],
        out_shape=[jax.ShapeDtypeStruct((N, 128), F32),
                   jax.ShapeDtypeStruct((N, 1), F32),
                   jax.ShapeDtypeStruct((1, 128), F32),
                   jax.ShapeDtypeStruct((1, 128), F32),
                   jax.ShapeDtypeStruct((1, 1), F32),
                   jax.ShapeDtypeStruct((1, 1), F32)],
    )(h, at, params['Wo'], params['bo'].reshape(1, 128),
      params['ln_g'].reshape(1, 128), params['ln_b'].reshape(1, 128),
      params['site_W1'], params['site_b1'].reshape(1, 64),
      params['site_W2'], params['site_b2'].reshape(1, 1), asg2d)


def _ppi_body(sA, sB, cA, cB, W1a, W1b, b1, W2, b2, o):
    ha = sA[...] / jnp.maximum(cA[...], 1.0)
    hb = sB[...] / jnp.maximum(cB[...], 1.0)
    z = jax.nn.relu(_dg(ha, W1a[...]) + _dg(hb, W1b[...]) + b1[...])
    o[...] = jax.nn.sigmoid(
        jnp.sum(z * W2[...], axis=1, keepdims=True) + b2[...])


def _ppi_call(sA, sB, cA, cB, params):
    full = lambda shp: pl.BlockSpec(shp, lambda: tuple(0 for _ in shp))
    return pl.pallas_call(
        _ppi_body,
        in_specs=[full((1, 128)), full((1, 128)), full((1, 1)), full((1, 1)),
                  full((128, 128)), full((128, 128)), full((1, 128)),
                  full((1, 128)), full((1, 1))],
        out_specs=full((1, 1)),
        out_shape=jax.ShapeDtypeStruct((1, 1), F32),
    )(sA, sB, cA, cB, params['ppi_W1'][:, :128], params['ppi_W1'][:, 128:],
      params['ppi_b1'].reshape(1, 128), params['ppi_W2'],
      params['ppi_b2'].reshape(1, 1))


_NW = 32
_CHUNK = 128


def _sc_mesh():
    return plsc.VectorSubcoreMesh(core_axis_name="c", subcore_axis_name="s")


def _sc_gather(Ga, Gb, row1d, col1d, E):
    nchunk = E // _CHUNK
    niter = (nchunk + _NW - 1) // _NW

    @functools.partial(
        pl.kernel, mesh=_sc_mesh(),
        out_type=[jax.ShapeDtypeStruct((E, 128), F32),
                  jax.ShapeDtypeStruct((E, 128), F32)],
        scratch_types=[pltpu.VMEM((_CHUNK,), jnp.int32),
                       pltpu.VMEM((_CHUNK,), jnp.int32),
                       pltpu.VMEM((_CHUNK, 256), F32),
                       pltpu.VMEM((_CHUNK, 256), F32),
                       pltpu.VMEM((_CHUNK, 128), F32),
                       pltpu.SemaphoreType.DMA,
                       pltpu.SemaphoreType.DMA],
    )
    def k(Ga_h, Gb_h, row_h, col_h, mpre_h, cdiff_h,
          ir, ic, bA, bB, bC, s1, s2):
        wid = lax.axis_index("s") * 2 + lax.axis_index("c")
        zv = jnp.zeros((16,), F32)

        @pl.loop(0, _CHUNK)
        def _(i):
            for c8 in range(1, 8):
                bC[i, pl.ds(c8 * 16, 16)] = zv

        @pl.loop(0, niter)
        def _(jj):
            r = wid + _NW * jj

            @pl.when(r < nchunk)
            def _():
                pltpu.sync_copy(row_h.at[pl.ds(r * _CHUNK, _CHUNK)], ir)
                pltpu.sync_copy(col_h.at[pl.ds(r * _CHUNK, _CHUNK)], ic)
                c1 = pltpu.async_copy(Ga_h.at[ir], bA, s1)
                c2 = pltpu.async_copy(Gb_h.at[ic], bB, s2)
                c1.wait()
                c2.wait()

                @pl.loop(0, _CHUNK)
                def _(i):
                    s0 = pl.ds(128, 16)
                    bC[i, pl.ds(0, 16)] = bA[i, s0] - bB[i, s0]
                    for c8 in range(8):
                        sl = pl.ds(c8 * 16, 16)
                        bA[i, sl] = bA[i, sl] + bB[i, sl]

                pltpu.sync_copy(bA.at[:, pl.ds(0, 128)],
                                mpre_h.at[pl.ds(r * _CHUNK, _CHUNK)])
                pltpu.sync_copy(bC, cdiff_h.at[pl.ds(r * _CHUNK, _CHUNK)])

    return k(Ga, Gb, row1d, col1d)


def _sc_scatter(msgs, cupd, row1d, z128, NP):
    E = msgs.shape[0]
    nchunk = E // _CHUNK
    niter = (nchunk + _NW - 1) // _NW
    rows_per_sub = NP // 16

    @functools.partial(
        pl.kernel, mesh=_sc_mesh(),
        out_type=[jax.ShapeDtypeStruct((2, NP, 128), F32),
                  jax.ShapeDtypeStruct((2, NP, 128), F32)],
        scratch_types=[pltpu.VMEM_SHARED((NP, 128), F32),
                       pltpu.VMEM((_CHUNK,), jnp.int32),
                       pltpu.VMEM((_CHUNK, 128), F32)],
    )
    def k(msgs_h, cupd_h, row_h, z128_h, agg_h, cagg_h, sh_acc, ir, mb):
        c = lax.axis_index("c")
        s = lax.axis_index("s")
        wid = s * 2 + c
        base = s * rows_per_sub
        for src_h, dst_h in ((msgs_h, agg_h), (cupd_h, cagg_h)):
            pltpu.sync_copy(z128_h.at[pl.ds(base, rows_per_sub)],
                            sh_acc.at[pl.ds(base, rows_per_sub)])
            plsc.subcore_barrier()

            @pl.loop(0, niter)
            def _(jj):
                r = wid + _NW * jj

                @pl.when(r < nchunk)
                def _():
                    pltpu.sync_copy(row_h.at[pl.ds(r * _CHUNK, _CHUNK)], ir)
                    pltpu.sync_copy(src_h.at[pl.ds(r * _CHUNK, _CHUNK)], mb)
                    pltpu.sync_copy(mb, sh_acc.at[ir], add=True)

            plsc.subcore_barrier()
            pltpu.sync_copy(sh_acc.at[pl.ds(base, rows_per_sub)],
                            dst_h.at[c, pl.ds(base, rows_per_sub)])
            plsc.subcore_barrier()

    return k(msgs, cupd, row1d, z128)


def kernel(node_features, coords, edges, protein_assignment, params):
    N, D = node_features.shape
    E = edges.shape[1]
    NP = ((N + 127) // 128) * 128
    row = edges[0].astype(jnp.int32)
    col = edges[1].astype(jnp.int32)
    cpad = jnp.zeros((N, 16), F32).at[:, :3].set(coords.astype(F32))
    z128 = jnp.zeros((NP, 128), F32)

    layers = params['layers']
    h, Ga, Gb = _emb_call(node_features, cpad, params['emb_W'],
                          params['emb_b'].reshape(1, 128),
                          layers[0]['msg_W1'][:, :128],
                          layers[0]['msg_W1'][:, 128:256], N, 1000)

    q = k_ = v = None
    for li in range(len(layers)):
        p = layers[li]
        mpre, cdiff = _sc_gather(Ga, Gb, row, col, E)
        msgs, cupd = _edge_call(mpre, cdiff, p, E, 8000)
        aggp, caggp = _sc_scatter(msgs, cupd, row, z128, NP)
        if li + 1 < len(layers):
            h, cpad, Ga, Gb = _node_call(h, aggp, cpad, caggp, p,
                                         layers[li + 1], N, 1000)
        else:
            h, q, k_, v = _node3_call(h, aggp, p, params, N, 1000)

    attn = _attn_call(q, k_, v, N, 1000)
    asg2d = protein_assignment.astype(jnp.int32).reshape(N, 1)
    hf, site, sA, sB, cA, cB = _final_call(h, attn, params, asg2d, N, 400)
    ppi = _ppi_call(sA, sB, cA, cB, params)
    return ppi.reshape(()), site.reshape(N), hf
